# baseline (speedup 1.0000x reference)
import functools

import jax
import jax.numpy as jnp
from jax import lax
from jax.experimental import pallas as pl
from jax.experimental.pallas import tpu as pltpu
from jax.experimental.pallas import tpu_sc as plsc

B = 16384
D = 128
L = 16
NSUB = D // L

NC = 2
NS = 16
NW = NC * NS
BPW = B // NW

CH = 128
NCH = BPW // CH
HH = CH // 2


def _transh_body(head_hbm, rel_hbm, tail_hbm, ent_hbm, relcat_hbm,
                 out_hbm, hidx, tidx, ridx, hbuf, tbuf, wrbuf, obuf,
                 gsem, wsem, osem):
    cid = lax.axis_index("c")
    sid = lax.axis_index("s")
    wid = sid * NC + cid
    base = wid * BPW

    pltpu.sync_copy(head_hbm.at[pl.ds(base, BPW)], hidx)
    pltpu.sync_copy(tail_hbm.at[pl.ds(base, BPW)], tidx)
    pltpu.sync_copy(rel_hbm.at[pl.ds(base, BPW)], ridx)

    def issue_ht(c, p):
        isl = pl.ds(c * CH, CH)
        return (
            pltpu.async_copy(ent_hbm.at[hidx.at[isl]], hbuf.at[p], gsem.at[p]),
            pltpu.async_copy(ent_hbm.at[tidx.at[isl]], tbuf.at[p], gsem.at[p]),
        )

    def issue_wr(c, half):
        isl = pl.ds(c * CH + half * HH, HH)
        return pltpu.async_copy(
            relcat_hbm.at[ridx.at[isl]], wrbuf.at[half], wsem.at[half])

    def compute_half(p, half):
        row0 = half * HH

        def row(i, carry):
            acc = jnp.zeros((L,), jnp.float32)
            hmts = []
            ws = []
            for j in range(NSUB):
                csl = pl.ds(j * L, L)
                h = hbuf[p, row0 + i, csl]
                t = tbuf[p, row0 + i, csl]
                w = wrbuf[half, i, csl]
                hmt = h - t
                acc = acc + hmt * w
                hmts.append(hmt)
                ws.append(w)
            d = jnp.sum(acc)
            for j in range(NSUB):
                csl = pl.ds(j * L, L)
                r = wrbuf[half, i, pl.ds(D + j * L, L)]
                obuf[row0 + i, csl] = hmts[j] + r - ws[j] * d
            return carry

        lax.fori_loop(0, HH, row, 0)

    gh = [None, None]
    gw = [None, None]
    oh = None
    gh[0] = issue_ht(0, 0)
    gw[0] = issue_wr(0, 0)
    gw[1] = issue_wr(0, 1)
    for c in range(NCH):
        p = c % 2
        if c + 1 < NCH:
            gh[1 - p] = issue_ht(c + 1, 1 - p)
        for h in gh[p]:
            h.wait()
        gw[0].wait()
        if oh is not None:
            oh.wait()
            oh = None
        compute_half(p, 0)
        if c + 1 < NCH:
            gw[0] = issue_wr(c + 1, 0)
        gw[1].wait()
        compute_half(p, 1)
        if c + 1 < NCH:
            gw[1] = issue_wr(c + 1, 1)
        oh = pltpu.async_copy(
            obuf, out_hbm.at[pl.ds(base + c * CH, CH)], osem)
    oh.wait()


_transh = functools.partial(
    pl.kernel,
    out_type=jax.ShapeDtypeStruct((B, D), jnp.float32),
    mesh=plsc.VectorSubcoreMesh(core_axis_name="c", subcore_axis_name="s"),
    compiler_params=pltpu.CompilerParams(needs_layout_passes=False),
    scratch_types=[
        pltpu.VMEM((BPW,), jnp.int32),
        pltpu.VMEM((BPW,), jnp.int32),
        pltpu.VMEM((BPW,), jnp.int32),
        pltpu.VMEM((2, CH, D), jnp.float32),
        pltpu.VMEM((2, CH, D), jnp.float32),
        pltpu.VMEM((2, HH, 2 * D), jnp.float32),
        pltpu.VMEM((CH, D), jnp.float32),
        pltpu.SemaphoreType.DMA((2,)),
        pltpu.SemaphoreType.DMA((2,)),
        pltpu.SemaphoreType.DMA,
    ],
)(_transh_body)


def kernel(head, relation, tail, ent_emb, rel_emb, rel_hyper):
    rel_cat = jnp.concatenate([rel_hyper, rel_emb], axis=1)
    return _transh(head, relation, tail, ent_emb, rel_cat)
